# baseline (speedup 1.0000x reference)
import jax
import jax.numpy as jnp
from jax import lax
from jax.experimental import pallas as pl
from jax.experimental.pallas import tpu as pltpu
from jax.experimental.pallas import tpu_sc as plsc

VOCAB = 1000000
DIM = 64
BATCH = 16384
NEG = 20

NC = 2
NS = 16
NW = NC * NS

BW = BATCH // NW
BWN = BW * NEG
CH = 256


def _sc_gather_body(cw_hbm, xw_hbm, nw_hbm, ie_hbm, oe_hbm,
                    outc_hbm, outx_hbm, outn_hbm,
                    idx_c, idx_x, idx_n, buf0, buf1, sem0, sem1):
    c = lax.axis_index("c")
    s = lax.axis_index("s")
    wid = s * NC + c
    b0 = wid * BW
    n0 = wid * BWN

    pltpu.sync_copy(cw_hbm.at[pl.ds(b0, BW)], idx_c)
    pltpu.sync_copy(xw_hbm.at[pl.ds(b0, BW)], idx_x)
    pltpu.sync_copy(nw_hbm.at[pl.ds(n0, BWN)], idx_n)

    def make_pair(table, idx_ref, out_hbm, base):
        def pair(j, carry):
            o0 = j * (2 * CH)
            cp0 = pltpu.async_copy(
                table.at[idx_ref.at[pl.ds(o0, CH)]], buf0, sem0)
            cp1 = pltpu.async_copy(
                table.at[idx_ref.at[pl.ds(o0 + CH, CH)]], buf1, sem1)
            cp0.wait()
            pltpu.sync_copy(buf0, out_hbm.at[pl.ds(base + o0, CH)])
            cp1.wait()
            pltpu.sync_copy(buf1, out_hbm.at[pl.ds(base + o0 + CH, CH)])
            return carry
        return pair

    lax.fori_loop(0, BW // (2 * CH), make_pair(ie_hbm, idx_c, outc_hbm, b0), 0)
    lax.fori_loop(0, BW // (2 * CH), make_pair(oe_hbm, idx_x, outx_hbm, b0), 0)
    lax.fori_loop(0, BWN // (2 * CH), make_pair(oe_hbm, idx_n, outn_hbm, n0), 0)


@jax.jit
def kernel(center_words, context_words, negative_words,
           input_embeddings, output_embeddings):
    mesh = plsc.VectorSubcoreMesh(core_axis_name="c", subcore_axis_name="s")
    run = pl.kernel(
        _sc_gather_body,
        mesh=mesh,
        compiler_params=pltpu.CompilerParams(use_tc_tiling_on_sc=False),
        out_type=[
            jax.ShapeDtypeStruct((BATCH, DIM), jnp.float32),
            jax.ShapeDtypeStruct((BATCH, DIM), jnp.float32),
            jax.ShapeDtypeStruct((BATCH * NEG, DIM), jnp.float32),
        ],
        scratch_types=[
            pltpu.VMEM((BW,), jnp.int32),
            pltpu.VMEM((BW,), jnp.int32),
            pltpu.VMEM((BWN,), jnp.int32),
            pltpu.VMEM((CH, DIM), jnp.float32),
            pltpu.VMEM((CH, DIM), jnp.float32),
            pltpu.SemaphoreType.DMA,
            pltpu.SemaphoreType.DMA,
        ],
    )
    center, context, neg_flat = run(
        center_words.astype(jnp.int32),
        context_words.astype(jnp.int32),
        negative_words.reshape(-1).astype(jnp.int32),
        input_embeddings,
        output_embeddings,
    )
    return center, context, neg_flat.reshape(BATCH, NEG, DIM)


# speedup vs baseline: 1.0073x; 1.0073x over previous
import jax
import jax.numpy as jnp
from jax import lax
from jax.experimental import pallas as pl
from jax.experimental.pallas import tpu as pltpu
from jax.experimental.pallas import tpu_sc as plsc

VOCAB = 1000000
DIM = 64
BATCH = 16384
NEG = 20

NC = 2
NS = 16
NW = NC * NS

BW = BATCH // NW
BWN = BW * NEG
CH = 256


K = 4
D = 2


def _sc_gather_body(cw_hbm, xw_hbm, nw_hbm, ie_hbm, oe_hbm,
                    outc_hbm, outx_hbm, outn_hbm,
                    idx_c, idx_x, idx_n, *scratch):
    bufs = scratch[:K]
    gsems = scratch[K:2 * K]
    wsems = scratch[2 * K:3 * K]
    c = lax.axis_index("c")
    s = lax.axis_index("s")
    wid = s * NC + c
    b0 = wid * BW
    n0 = wid * BWN

    pltpu.sync_copy(cw_hbm.at[pl.ds(b0, BW)], idx_c)
    pltpu.sync_copy(xw_hbm.at[pl.ds(b0, BW)], idx_x)
    pltpu.sync_copy(nw_hbm.at[pl.ds(n0, BWN)], idx_n)

    chunks = []
    for j in range(BW // CH):
        chunks.append((ie_hbm, idx_c, j * CH, outc_hbm, b0 + j * CH))
    for j in range(BW // CH):
        chunks.append((oe_hbm, idx_x, j * CH, outx_hbm, b0 + j * CH))
    for j in range(BWN // CH):
        chunks.append((oe_hbm, idx_n, j * CH, outn_hbm, n0 + j * CH))
    n = len(chunks)

    gcp = [None] * K
    wcp = [None] * K
    for i in range(n + D):
        if i < n:
            k = i % K
            if wcp[k] is not None:
                wcp[k].wait()
            tbl, idx, io, _, _ = chunks[i]
            gcp[k] = pltpu.async_copy(
                tbl.at[idx.at[pl.ds(io, CH)]], bufs[k], gsems[k])
        if i >= D:
            j = i - D
            k = j % K
            gcp[k].wait()
            _, _, _, out, oo = chunks[j]
            wcp[k] = pltpu.async_copy(
                bufs[k], out.at[pl.ds(oo, CH)], wsems[k])
    for k in range(K):
        if wcp[k] is not None:
            wcp[k].wait()


@jax.jit
def kernel(center_words, context_words, negative_words,
           input_embeddings, output_embeddings):
    mesh = plsc.VectorSubcoreMesh(core_axis_name="c", subcore_axis_name="s")
    run = pl.kernel(
        _sc_gather_body,
        mesh=mesh,
        compiler_params=pltpu.CompilerParams(use_tc_tiling_on_sc=False),
        out_type=[
            jax.ShapeDtypeStruct((BATCH, DIM), jnp.float32),
            jax.ShapeDtypeStruct((BATCH, DIM), jnp.float32),
            jax.ShapeDtypeStruct((BATCH * NEG, DIM), jnp.float32),
        ],
        scratch_types=(
            [pltpu.VMEM((BW,), jnp.int32),
             pltpu.VMEM((BW,), jnp.int32),
             pltpu.VMEM((BWN,), jnp.int32)]
            + [pltpu.VMEM((CH, DIM), jnp.float32)] * K
            + [pltpu.SemaphoreType.DMA] * (2 * K)
        ),
    )
    center, context, neg_flat = run(
        center_words.astype(jnp.int32),
        context_words.astype(jnp.int32),
        negative_words.reshape(-1).astype(jnp.int32),
        input_embeddings,
        output_embeddings,
    )
    return center, context, neg_flat.reshape(BATCH, NEG, DIM)


# speedup vs baseline: 1.0267x; 1.0192x over previous
import jax
import jax.numpy as jnp
from jax import lax
from jax.experimental import pallas as pl
from jax.experimental.pallas import tpu as pltpu
from jax.experimental.pallas import tpu_sc as plsc

VOCAB = 1000000
DIM = 64
BATCH = 16384
NEG = 20

NC = 2
NS = 16
NW = NC * NS

BW = BATCH // NW
BWN = BW * NEG
CH = 256
K = 4
D = 2


def _worker_base():
    c = lax.axis_index("c")
    s = lax.axis_index("s")
    return s * NC + c


def _pipelined_gather(chunks, bufs, gsems, wsems):
    n = len(chunks)
    gcp = [None] * K
    wcp = [None] * K
    for i in range(n + D):
        if i < n:
            k = i % K
            if wcp[k] is not None:
                wcp[k].wait()
            tbl, idx, io, _, _ = chunks[i]
            gcp[k] = pltpu.async_copy(
                tbl.at[idx.at[pl.ds(io, CH)]], bufs[k], gsems[k])
        if i >= D:
            j = i - D
            k = j % K
            gcp[k].wait()
            _, _, _, out, oo = chunks[j]
            wcp[k] = pltpu.async_copy(
                bufs[k], out.at[pl.ds(oo, CH)], wsems[k])
    for k in range(K):
        if wcp[k] is not None:
            wcp[k].wait()


def _center_body(cw_hbm, ie_hbm, outc_hbm, idx_c, *scratch):
    bufs = scratch[:K]
    gsems = scratch[K:2 * K]
    wsems = scratch[2 * K:3 * K]
    b0 = _worker_base() * BW
    pltpu.sync_copy(cw_hbm.at[pl.ds(b0, BW)], idx_c)
    chunks = [(ie_hbm, idx_c, j * CH, outc_hbm, b0 + j * CH)
              for j in range(BW // CH)]
    _pipelined_gather(chunks, bufs, gsems, wsems)


def _context_neg_body(xw_hbm, nw_hbm, oe_hbm, outx_hbm, outn_hbm,
                      idx_x, idx_n, *scratch):
    bufs = scratch[:K]
    gsems = scratch[K:2 * K]
    wsems = scratch[2 * K:3 * K]
    wid = _worker_base()
    b0 = wid * BW
    n0 = wid * BWN
    pltpu.sync_copy(xw_hbm.at[pl.ds(b0, BW)], idx_x)
    pltpu.sync_copy(nw_hbm.at[pl.ds(n0, BWN)], idx_n)
    chunks = [(oe_hbm, idx_x, j * CH, outx_hbm, b0 + j * CH)
              for j in range(BW // CH)]
    chunks += [(oe_hbm, idx_n, j * CH, outn_hbm, n0 + j * CH)
               for j in range(BWN // CH)]
    _pipelined_gather(chunks, bufs, gsems, wsems)


_BUF_SCRATCH = (
    [pltpu.VMEM((CH, DIM), jnp.float32)] * K
    + [pltpu.SemaphoreType.DMA] * (2 * K)
)


@jax.jit
def kernel(center_words, context_words, negative_words,
           input_embeddings, output_embeddings):
    mesh = plsc.VectorSubcoreMesh(core_axis_name="c", subcore_axis_name="s")
    params = pltpu.CompilerParams(use_tc_tiling_on_sc=False)

    run_center = pl.kernel(
        _center_body,
        mesh=mesh,
        compiler_params=params,
        out_type=jax.ShapeDtypeStruct((BATCH, DIM), jnp.float32),
        scratch_types=[pltpu.VMEM((BW,), jnp.int32)] + _BUF_SCRATCH,
    )
    run_ctx_neg = pl.kernel(
        _context_neg_body,
        mesh=mesh,
        compiler_params=params,
        out_type=[
            jax.ShapeDtypeStruct((BATCH, DIM), jnp.float32),
            jax.ShapeDtypeStruct((BATCH * NEG, DIM), jnp.float32),
        ],
        scratch_types=[pltpu.VMEM((BW,), jnp.int32),
                       pltpu.VMEM((BWN,), jnp.int32)] + _BUF_SCRATCH,
    )

    center = run_center(center_words.astype(jnp.int32), input_embeddings)
    context, neg_flat = run_ctx_neg(
        context_words.astype(jnp.int32),
        negative_words.reshape(-1).astype(jnp.int32),
        output_embeddings,
    )
    return center, context, neg_flat.reshape(BATCH, NEG, DIM)
